# Initial kernel scaffold; baseline (speedup 1.0000x reference)
import jax
import jax.numpy as jnp
from jax.experimental import pallas as pl


def kernel(pos, batch, params):
    raise NotImplementedError("write your pallas kernel here")


# speedup vs baseline: 4.8191x; 4.8191x over previous
import functools

import jax
import jax.numpy as jnp
from jax import lax
from jax.experimental import pallas as pl
from jax.experimental.pallas import tpu as pltpu
from jax.experimental.pallas import tpu_sc as plsc

_N = 8192
_K = 20
_G = 16
_BIGMASK = 1e30
_NEG = -1e30
_POS = 1e30
_IT = False


def _knn(x, batch_col, batch_rowT):
    N, F = x.shape
    BR = 128

    def body(xr_ref, bc_ref, xall_ref, brT_ref, idx_ref, d_scr):
        xr = xr_ref[:]
        xa = xall_ref[:]
        sqr = jnp.sum(xr * xr, axis=1, keepdims=True)
        sqcT = lax.dot_general(
            jnp.ones((1, F), jnp.float32), xa * xa,
            (((1,), (1,)), ((), ())),
            preferred_element_type=jnp.float32,
            precision=lax.Precision.HIGHEST)
        prod = lax.dot_general(
            xr, xa, (((1,), (1,)), ((), ())),
            preferred_element_type=jnp.float32,
            precision=lax.Precision.DEFAULT)
        d = sqr + sqcT - 2.0 * prod
        d = jnp.where(bc_ref[:] != brT_ref[:], _BIGMASK, d)
        d_scr[:] = d
        colio = lax.broadcasted_iota(jnp.int32, (BR, N), 1)
        for t in range(_K):
            dd = d_scr[:]
            m = jnp.min(dd, axis=1, keepdims=True)
            cand = jnp.where(dd == m, colio, 2**30)
            amin = jnp.min(cand, axis=1, keepdims=True)
            idx_ref[:, t:t + 1] = amin
            d_scr[:] = jnp.where(colio == amin, float("inf"), dd)

    return pl.pallas_call(
        body,
        grid=(N // BR,),
        in_specs=[
            pl.BlockSpec((BR, F), lambda i: (i, 0)),
            pl.BlockSpec((BR, 1), lambda i: (i, 0)),
            pl.BlockSpec((N, F), lambda i: (0, 0)),
            pl.BlockSpec((1, N), lambda i: (0, 0)),
        ],
        out_specs=pl.BlockSpec((BR, _K), lambda i: (i, 0)),
        out_shape=jax.ShapeDtypeStruct((N, _K), jnp.int32),
        scratch_shapes=[pltpu.VMEM((BR, N), jnp.float32)],
        interpret=_IT,
    )(x, batch_col, x, batch_rowT)


def _sc_gather(table, idx):
    V, D = table.shape
    E = idx.shape[0]
    NC, NS = 2, 16
    NW = NC * NS
    per = E // NW
    CH = 128
    nch = per // CH
    mesh = plsc.VectorSubcoreMesh(core_axis_name="c", subcore_axis_name="s",
                                  num_cores=NC, num_subcores=NS)

    @functools.partial(
        pl.kernel, mesh=mesh,
        out_type=jax.ShapeDtypeStruct((E, D), jnp.float32),
        scratch_types=[
            pltpu.VMEM((CH,), jnp.int32),
            pltpu.VMEM((CH, D), jnp.float32),
            pltpu.SemaphoreType.DMA,
        ],
        compiler_params=pltpu.CompilerParams(use_tc_tiling_on_sc=False),
        interpret=_IT,
    )
    def k(table_hbm, idx_hbm, out_hbm, idx_v, rows_v, sem):
        wid = lax.axis_index("s") * NC + lax.axis_index("c")
        base = wid * per

        def chunk(c, carry):
            off = base + c * CH
            pltpu.sync_copy(idx_hbm.at[pl.ds(off, CH)], idx_v)
            pltpu.async_copy(table_hbm.at[idx_v], rows_v, sem).wait()
            pltpu.sync_copy(rows_v, out_hbm.at[pl.ds(off, CH)])
            return carry

        lax.fori_loop(0, nch, chunk, 0)

    return k(table, idx)


def _edge_l1(x, xj3, W, b, want_h, want_minmax):
    N, F = x.shape
    C = W.shape[1]
    BR = 128
    nb = N // BR

    def body(x_ref, xj_ref, w_ref, b_ref, *refs):
        r = list(refs)
        h_ref = r.pop(0) if want_h else None
        if want_minmax:
            mx_ref = r.pop(0)
            mn_ref = r.pop(0)
        sum_ref, sq_ref = r
        xi = x_ref[:]
        w = w_ref[:]
        bb = b_ref[:]
        s = jnp.zeros((BR, C), jnp.float32)
        s2 = jnp.zeros((BR, C), jnp.float32)
        mx = mn = None
        for kk in range(_K):
            e = jnp.concatenate([xi, xj_ref[:, kk, :] - xi], axis=1)
            h = jnp.maximum(
                jnp.dot(e, w, preferred_element_type=jnp.float32,
                        precision=lax.Precision.DEFAULT) + bb, 0.0)
            if want_h:
                h_ref[:, kk, :] = h
            if want_minmax:
                mx = h if kk == 0 else jnp.maximum(mx, h)
                mn = h if kk == 0 else jnp.minimum(mn, h)
            s = s + h
            s2 = s2 + h * h
        if want_minmax:
            mx_ref[:] = mx
            mn_ref[:] = mn
        sum_ref[0] = jnp.sum(s, axis=0, keepdims=True)
        sq_ref[0] = jnp.sum(s2, axis=0, keepdims=True)

    out_specs = []
    out_shapes = []
    if want_h:
        out_specs.append(pl.BlockSpec((BR, _K, C), lambda i: (i, 0, 0)))
        out_shapes.append(jax.ShapeDtypeStruct((N, _K, C), jnp.float32))
    if want_minmax:
        for _ in range(2):
            out_specs.append(pl.BlockSpec((BR, C), lambda i: (i, 0)))
            out_shapes.append(jax.ShapeDtypeStruct((N, C), jnp.float32))
    for _ in range(2):
        out_specs.append(pl.BlockSpec((1, 1, C), lambda i: (i, 0, 0)))
        out_shapes.append(jax.ShapeDtypeStruct((nb, 1, C), jnp.float32))

    return pl.pallas_call(
        body,
        grid=(nb,),
        in_specs=[
            pl.BlockSpec((BR, F), lambda i: (i, 0)),
            pl.BlockSpec((BR, _K, F), lambda i: (i, 0, 0)),
            pl.BlockSpec(W.shape, lambda i: (0, 0)),
            pl.BlockSpec((1, C), lambda i: (0, 0)),
        ],
        out_specs=out_specs,
        out_shape=out_shapes,
        interpret=_IT,
    )(x, xj3, W, b)


def _dense_relu(X, a, c, W, b, want_minmax):
    if want_minmax:
        N, K, Ci = X.shape
        BR = 128
    else:
        R, Ci = X.shape
        BR = 2048
        nb = R // BR
    Co = W.shape[1]

    if want_minmax:
        nb = N // BR

        def body(x_ref, a_ref, c_ref, w_ref, b_ref,
                 mx_ref, mn_ref, sum_ref, sq_ref):
            aa = a_ref[:]
            cc = c_ref[:]
            w = w_ref[:]
            bb = b_ref[:]
            s = jnp.zeros((BR, Co), jnp.float32)
            s2 = jnp.zeros((BR, Co), jnp.float32)
            mx = mn = None
            for kk in range(K):
                xn = x_ref[:, kk, :] * aa + cc
                h = jnp.maximum(
                    jnp.dot(xn, w, preferred_element_type=jnp.float32,
                            precision=lax.Precision.DEFAULT) + bb, 0.0)
                mx = h if kk == 0 else jnp.maximum(mx, h)
                mn = h if kk == 0 else jnp.minimum(mn, h)
                s = s + h
                s2 = s2 + h * h
            mx_ref[:] = mx
            mn_ref[:] = mn
            sum_ref[0] = jnp.sum(s, axis=0, keepdims=True)
            sq_ref[0] = jnp.sum(s2, axis=0, keepdims=True)

        return pl.pallas_call(
            body,
            grid=(nb,),
            in_specs=[
                pl.BlockSpec((BR, K, Ci), lambda i: (i, 0, 0)),
                pl.BlockSpec((1, Ci), lambda i: (0, 0)),
                pl.BlockSpec((1, Ci), lambda i: (0, 0)),
                pl.BlockSpec(W.shape, lambda i: (0, 0)),
                pl.BlockSpec((1, Co), lambda i: (0, 0)),
            ],
            out_specs=[
                pl.BlockSpec((BR, Co), lambda i: (i, 0)),
                pl.BlockSpec((BR, Co), lambda i: (i, 0)),
                pl.BlockSpec((1, 1, Co), lambda i: (i, 0, 0)),
                pl.BlockSpec((1, 1, Co), lambda i: (i, 0, 0)),
            ],
            out_shape=[
                jax.ShapeDtypeStruct((N, Co), jnp.float32),
                jax.ShapeDtypeStruct((N, Co), jnp.float32),
                jax.ShapeDtypeStruct((nb, 1, Co), jnp.float32),
                jax.ShapeDtypeStruct((nb, 1, Co), jnp.float32),
            ],
            interpret=_IT,
        )(X, a, c, W, b)

    def body(x_ref, a_ref, c_ref, w_ref, b_ref, h_ref, sum_ref, sq_ref):
        xn = x_ref[:] * a_ref[:] + c_ref[:]
        h = jnp.maximum(
            jnp.dot(xn, w_ref[:], preferred_element_type=jnp.float32,
                    precision=lax.Precision.DEFAULT) + b_ref[:], 0.0)
        h_ref[:] = h
        sum_ref[0] = jnp.sum(h, axis=0, keepdims=True)
        sq_ref[0] = jnp.sum(h * h, axis=0, keepdims=True)

    return pl.pallas_call(
        body,
        grid=(nb,),
        in_specs=[
            pl.BlockSpec((BR, Ci), lambda i: (i, 0)),
            pl.BlockSpec((1, Ci), lambda i: (0, 0)),
            pl.BlockSpec((1, Ci), lambda i: (0, 0)),
            pl.BlockSpec(W.shape, lambda i: (0, 0)),
            pl.BlockSpec((1, Co), lambda i: (0, 0)),
        ],
        out_specs=[
            pl.BlockSpec((BR, Co), lambda i: (i, 0)),
            pl.BlockSpec((1, 1, Co), lambda i: (i, 0, 0)),
            pl.BlockSpec((1, 1, Co), lambda i: (i, 0, 0)),
        ],
        out_shape=[
            jax.ShapeDtypeStruct((R, Co), jnp.float32),
            jax.ShapeDtypeStruct((nb, 1, Co), jnp.float32),
            jax.ShapeDtypeStruct((nb, 1, Co), jnp.float32),
        ],
        interpret=_IT,
    )(X, a, c, W, b)


def _affine_sel(mx, mn, a, c):
    N, C = mx.shape
    BR = 1024

    def body(mx_ref, mn_ref, a_ref, c_ref, o_ref):
        a = a_ref[:]
        o_ref[:] = jnp.where(a > 0, a * mx_ref[:], a * mn_ref[:]) + c_ref[:]

    return pl.pallas_call(
        body,
        grid=(N // BR,),
        in_specs=[
            pl.BlockSpec((BR, C), lambda i: (i, 0)),
            pl.BlockSpec((BR, C), lambda i: (i, 0)),
            pl.BlockSpec((1, C), lambda i: (0, 0)),
            pl.BlockSpec((1, C), lambda i: (0, 0)),
        ],
        out_specs=pl.BlockSpec((BR, C), lambda i: (i, 0)),
        out_shape=jax.ShapeDtypeStruct((N, C), jnp.float32),
        interpret=_IT,
    )(mx, mn, a, c)


def _lin1(x1, x2, batch_col, Wa, Wb, b):
    N = x1.shape[0]
    C = Wa.shape[1]
    BR = 256
    nb = N // BR

    def body(x1_ref, x2_ref, bat_ref, wa_ref, wb_ref, b_ref,
             sum_ref, sq_ref, mx_ref, mn_ref, mxs, mns):
        h = jnp.dot(x1_ref[:], wa_ref[:], preferred_element_type=jnp.float32,
                    precision=lax.Precision.DEFAULT)
        h = h + jnp.dot(x2_ref[:], wb_ref[:],
                        preferred_element_type=jnp.float32,
                        precision=lax.Precision.DEFAULT)
        h = jnp.maximum(h + b_ref[:], 0.0)
        sum_ref[0] = jnp.sum(h, axis=0, keepdims=True)
        sq_ref[0] = jnp.sum(h * h, axis=0, keepdims=True)
        bat = bat_ref[:]
        for g in range(_G):
            m = bat == g
            mxs[g:g + 1, :] = jnp.max(jnp.where(m, h, _NEG), axis=0,
                                      keepdims=True)
            mns[g:g + 1, :] = jnp.min(jnp.where(m, h, _POS), axis=0,
                                      keepdims=True)
        mx_ref[0] = mxs[:]
        mn_ref[0] = mns[:]

    return pl.pallas_call(
        body,
        grid=(nb,),
        in_specs=[
            pl.BlockSpec((BR, x1.shape[1]), lambda i: (i, 0)),
            pl.BlockSpec((BR, x2.shape[1]), lambda i: (i, 0)),
            pl.BlockSpec((BR, 1), lambda i: (i, 0)),
            pl.BlockSpec(Wa.shape, lambda i: (0, 0)),
            pl.BlockSpec(Wb.shape, lambda i: (0, 0)),
            pl.BlockSpec((1, C), lambda i: (0, 0)),
        ],
        out_specs=[
            pl.BlockSpec((1, 1, C), lambda i: (i, 0, 0)),
            pl.BlockSpec((1, 1, C), lambda i: (i, 0, 0)),
            pl.BlockSpec((1, _G, C), lambda i: (i, 0, 0)),
            pl.BlockSpec((1, _G, C), lambda i: (i, 0, 0)),
        ],
        out_shape=[
            jax.ShapeDtypeStruct((nb, 1, C), jnp.float32),
            jax.ShapeDtypeStruct((nb, 1, C), jnp.float32),
            jax.ShapeDtypeStruct((nb, _G, C), jnp.float32),
            jax.ShapeDtypeStruct((nb, _G, C), jnp.float32),
        ],
        scratch_shapes=[pltpu.VMEM((_G, C), jnp.float32),
                        pltpu.VMEM((_G, C), jnp.float32)],
        interpret=_IT,
    )(x1, x2, batch_col, Wa, Wb, b)


def _tail(mxP, mnP, sumP, sqP, g5, be5, p6, p7, Wf, bf):
    nb, G, C = mxP.shape
    n = float(_N)

    def body(mxP_ref, mnP_ref, sumP_ref, sqP_ref, g5_ref, be5_ref,
             W6_ref, b6_ref, g6_ref, be6_ref,
             W7_ref, b7_ref, g7_ref, be7_ref, Wf_ref, bf_ref, o_ref):
        s = sumP_ref[0]
        s2 = sqP_ref[0]
        MX = mxP_ref[0]
        MN = mnP_ref[0]
        for i in range(1, nb):
            s = s + sumP_ref[i]
            s2 = s2 + sqP_ref[i]
            MX = jnp.maximum(MX, mxP_ref[i])
            MN = jnp.minimum(MN, mnP_ref[i])
        m = s / n
        v = s2 / n - m * m
        aL = g5_ref[:] / jnp.sqrt(v + 1e-5)
        bL = be5_ref[:] - m * aL
        pooled = jnp.where(aL > 0, aL * MX, aL * MN) + bL

        def block(x, W_ref, b_ref, g_ref, be_ref):
            h = jnp.maximum(
                jnp.dot(x, W_ref[:], preferred_element_type=jnp.float32,
                        precision=lax.Precision.DEFAULT) + b_ref[:], 0.0)
            mu = jnp.mean(h, axis=0, keepdims=True)
            va = jnp.mean((h - mu) * (h - mu), axis=0, keepdims=True)
            return (h - mu) / jnp.sqrt(va + 1e-5) * g_ref[:] + be_ref[:]

        h1 = block(pooled, W6_ref, b6_ref, g6_ref, be6_ref)
        h2 = block(h1, W7_ref, b7_ref, g7_ref, be7_ref)
        logits = jnp.dot(h2, Wf_ref[:], preferred_element_type=jnp.float32,
                         precision=lax.Precision.DEFAULT) + bf_ref[:]
        z = logits - jnp.max(logits, axis=1, keepdims=True)
        o_ref[:] = z - jnp.log(jnp.sum(jnp.exp(z), axis=1, keepdims=True))

    ins = [mxP, mnP, sumP, sqP, g5, be5,
           p6["W"], p6["b"][None, :], p6["g"][None, :], p6["be"][None, :],
           p7["W"], p7["b"][None, :], p7["g"][None, :], p7["be"][None, :],
           Wf, bf]
    in_specs = [pl.BlockSpec(a.shape, lambda i, nd=a.ndim: (0,) * nd)
                for a in ins]

    return pl.pallas_call(
        body,
        grid=(1,),
        in_specs=in_specs,
        out_specs=pl.BlockSpec((G, 40), lambda i: (0, 0)),
        out_shape=jax.ShapeDtypeStruct((G, 40), jnp.float32),
        interpret=_IT,
    )(*ins)


def _bn_affine(sumP, sqP, n, g, be):
    s = jnp.sum(sumP, axis=(0, 1))
    s2 = jnp.sum(sqP, axis=(0, 1))
    m = s / n
    v = s2 / n - m * m
    a = g / jnp.sqrt(v + 1e-5)
    return a[None, :], (be - m * a)[None, :]


def kernel(pos, batch, params):
    N = pos.shape[0]
    batch_col = batch.astype(jnp.int32).reshape(N, 1)
    batch_rowT = batch.astype(jnp.int32).reshape(1, N)
    nE = jnp.float32(N * _K)

    pos16 = jnp.concatenate([pos, jnp.zeros((N, 13), jnp.float32)], axis=1)
    idx1 = _knn(pos16[:, :8], batch_col, batch_rowT)
    c1 = params["conv1"]
    W1p = jnp.zeros((32, 64), jnp.float32)
    W1p = W1p.at[0:3].set(c1[0]["W"][:3]).at[16:19].set(c1[0]["W"][3:])
    posj = _sc_gather(pos16, idx1.reshape(-1))
    H1, s1, q1 = _edge_l1(pos16, posj.reshape(N, _K, -1), W1p,
                          c1[0]["b"][None, :], want_h=True, want_minmax=False)
    a1, b1 = _bn_affine(s1, q1, nE, c1[0]["g"], c1[0]["be"])
    H2, s2, q2 = _dense_relu(H1.reshape(N * _K, -1), a1, b1,
                             c1[1]["W"], c1[1]["b"][None, :],
                             want_minmax=False)
    a2, b2 = _bn_affine(s2, q2, nE, c1[1]["g"], c1[1]["be"])
    mx1, mn1, s3, q3 = _dense_relu(H2.reshape(N, _K, -1), a2, b2,
                                   c1[2]["W"], c1[2]["b"][None, :],
                                   want_minmax=True)
    a3, b3 = _bn_affine(s3, q3, nE, c1[2]["g"], c1[2]["be"])
    x1 = _affine_sel(mx1, mn1, a3, b3)

    idx2 = _knn(x1, batch_col, batch_rowT)
    c2p = params["conv2"][0]
    x1j = _sc_gather(x1, idx2.reshape(-1))
    mx2, mn2, s4, q4 = _edge_l1(x1, x1j.reshape(N, _K, -1), c2p["W"],
                                c2p["b"][None, :],
                                want_h=False, want_minmax=True)
    a4, b4 = _bn_affine(s4, q4, nE, c2p["g"], c2p["be"])
    x2 = _affine_sel(mx2, mn2, a4, b4)

    l1 = params["lin1"][0]
    Wa, Wb = l1["W"][:64], l1["W"][64:]
    sumP, sqP, mxP, mnP = _lin1(x1, x2, batch_col, Wa, Wb, l1["b"][None, :])
    return _tail(mxP, mnP, sumP, sqP, l1["g"][None, :], l1["be"][None, :],
                 params["mlp1"][0], params["mlp2"][0],
                 params["final"]["W"], params["final"]["b"][None, :])


# speedup vs baseline: 5.9230x; 1.2291x over previous
import functools

import jax
import jax.numpy as jnp
from jax import lax
from jax.experimental import pallas as pl
from jax.experimental.pallas import tpu as pltpu
from jax.experimental.pallas import tpu_sc as plsc

_N = 8192
_K = 20
_G = 16
_BIGMASK = 1e30
_NEG = -1e30
_POS = 1e30
_IT = False


def _knn(x, batch_col, batch_rowT, lohi):
    N, F = x.shape
    BR = 128
    W = 1024
    NS = N // W
    nb = N // BR

    def body(lohi_ref, xr_ref, bc_ref, xall_ref, brT_ref, idx_ref,
             dwin, cd, ci):
        i = pl.program_id(0)
        lo = lohi_ref[i, 0]
        hi = lohi_ref[i, 1]
        nwin = (hi - lo + (W - 1)) // W
        xr = xr_ref[:]
        sqr = jnp.sum(xr * xr, axis=1, keepdims=True)
        bc = bc_ref[:]
        cd[:] = jnp.full((NS * BR, 32), float("inf"), jnp.float32)

        def win_body(w, carry):
            lo_w = pl.multiple_of(jnp.minimum(lo + w * W, N - W), 512)
            xw = xall_ref[pl.ds(lo_w, W), :]
            sqw = lax.dot_general(
                jnp.ones((1, F), jnp.float32), xw * xw,
                (((1,), (1,)), ((), ())),
                preferred_element_type=jnp.float32,
                precision=lax.Precision.HIGHEST)
            pw = lax.dot_general(
                xr, xw, (((1,), (1,)), ((), ())),
                preferred_element_type=jnp.float32,
                precision=lax.Precision.DEFAULT)
            dw = sqr + sqw - 2.0 * pw
            bw = brT_ref[:, pl.ds(lo_w, W)]
            dw = jnp.where(bc != bw, _BIGMASK, dw)
            io = lax.broadcasted_iota(jnp.int32, (BR, W), 1) + lo_w
            dwin[:] = dw
            for t in range(_K):
                dd = dwin[:]
                m = jnp.min(dd, axis=1, keepdims=True)
                amin = jnp.min(jnp.where(dd == m, io, 2**30),
                               axis=1, keepdims=True)
                cd[pl.ds(w * BR, BR), t:t + 1] = m
                ci[pl.ds(w * BR, BR), t:t + 1] = amin
                dwin[:] = jnp.where(io == amin, float("inf"), dd)
            return carry

        lax.fori_loop(0, nwin, win_body, 0)

        sd = [cd[w * BR:(w + 1) * BR, :] for w in range(NS)]
        si = [ci[w * BR:(w + 1) * BR, :] for w in range(NS)]
        for t in range(_K):
            m = jnp.min(sd[0], axis=1, keepdims=True)
            for w in range(1, NS):
                m = jnp.minimum(m, jnp.min(sd[w], axis=1, keepdims=True))
            am = jnp.min(jnp.where(sd[0] == m, si[0], 2**30),
                         axis=1, keepdims=True)
            for w in range(1, NS):
                am = jnp.minimum(
                    am, jnp.min(jnp.where(sd[w] == m, si[w], 2**30),
                                axis=1, keepdims=True))
            idx_ref[:, t:t + 1] = am
            sd = [jnp.where(si[w] == am, float("inf"), sd[w])
                  for w in range(NS)]

    return pl.pallas_call(
        body,
        grid=(nb,),
        in_specs=[
            pl.BlockSpec(memory_space=pltpu.SMEM),
            pl.BlockSpec((BR, F), lambda i: (i, 0)),
            pl.BlockSpec((BR, 1), lambda i: (i, 0)),
            pl.BlockSpec((N, F), lambda i: (0, 0)),
            pl.BlockSpec((1, N), lambda i: (0, 0)),
        ],
        out_specs=pl.BlockSpec((BR, _K), lambda i: (i, 0)),
        out_shape=jax.ShapeDtypeStruct((N, _K), jnp.int32),
        scratch_shapes=[pltpu.VMEM((BR, W), jnp.float32),
                        pltpu.VMEM((NS * BR, 32), jnp.float32),
                        pltpu.VMEM((NS * BR, 32), jnp.int32)],
        interpret=_IT,
    )(lohi, x, batch_col, x, batch_rowT)


def _knn_ranges(batch, N, BR=128):
    b = batch.astype(jnp.int32)
    ga = jnp.arange(_G, dtype=jnp.int32)
    starts = jnp.searchsorted(b, ga, side="left").astype(jnp.int32)
    ends = jnp.searchsorted(b, ga, side="right").astype(jnp.int32)
    counts = ends - starts
    g_lo = b[0::BR]
    g_hi = b[BR - 1::BR]
    lo = starts[g_lo]
    hi = ends[g_hi]
    span = (ga[None, :] >= g_lo[:, None]) & (ga[None, :] <= g_hi[:, None])
    mc = jnp.min(jnp.where(span, counts[None, :], 2**30), axis=1)
    small = mc < _K
    lo = jnp.where(small, 0, lo) & ~511
    hi = jnp.where(small, N, hi)
    return jnp.stack([lo, hi], axis=1).astype(jnp.int32)


def _sc_gather(table, idx):
    V, D = table.shape
    E = idx.shape[0]
    NC, NS = 2, 16
    NW = NC * NS
    per = E // NW
    CH = 128
    nch = per // CH
    mesh = plsc.VectorSubcoreMesh(core_axis_name="c", subcore_axis_name="s",
                                  num_cores=NC, num_subcores=NS)

    @functools.partial(
        pl.kernel, mesh=mesh,
        out_type=jax.ShapeDtypeStruct((E, D), jnp.float32),
        scratch_types=[
            pltpu.VMEM((CH,), jnp.int32),
            pltpu.VMEM((CH, D), jnp.float32),
            pltpu.SemaphoreType.DMA,
        ],
        compiler_params=pltpu.CompilerParams(use_tc_tiling_on_sc=False),
        interpret=_IT,
    )
    def k(table_hbm, idx_hbm, out_hbm, idx_v, rows_v, sem):
        wid = lax.axis_index("s") * NC + lax.axis_index("c")
        base = wid * per

        def chunk(c, carry):
            off = base + c * CH
            pltpu.sync_copy(idx_hbm.at[pl.ds(off, CH)], idx_v)
            pltpu.async_copy(table_hbm.at[idx_v], rows_v, sem).wait()
            pltpu.sync_copy(rows_v, out_hbm.at[pl.ds(off, CH)])
            return carry

        lax.fori_loop(0, nch, chunk, 0)

    return k(table, idx)


def _edge_l1(x, xj3, W, b, want_h, want_minmax):
    N, F = x.shape
    C = W.shape[1]
    BR = 128
    nb = N // BR

    def body(x_ref, xj_ref, w_ref, b_ref, *refs):
        r = list(refs)
        h_ref = r.pop(0) if want_h else None
        if want_minmax:
            mx_ref = r.pop(0)
            mn_ref = r.pop(0)
        sum_ref, sq_ref = r
        xi = x_ref[:]
        w = w_ref[:]
        bb = b_ref[:]
        s = jnp.zeros((BR, C), jnp.float32)
        s2 = jnp.zeros((BR, C), jnp.float32)
        mx = mn = None
        for kk in range(_K):
            e = jnp.concatenate([xi, xj_ref[:, kk, :] - xi], axis=1)
            h = jnp.maximum(
                jnp.dot(e, w, preferred_element_type=jnp.float32,
                        precision=lax.Precision.DEFAULT) + bb, 0.0)
            if want_h:
                h_ref[:, kk, :] = h
            if want_minmax:
                mx = h if kk == 0 else jnp.maximum(mx, h)
                mn = h if kk == 0 else jnp.minimum(mn, h)
            s = s + h
            s2 = s2 + h * h
        if want_minmax:
            mx_ref[:] = mx
            mn_ref[:] = mn
        sum_ref[0] = jnp.sum(s, axis=0, keepdims=True)
        sq_ref[0] = jnp.sum(s2, axis=0, keepdims=True)

    out_specs = []
    out_shapes = []
    if want_h:
        out_specs.append(pl.BlockSpec((BR, _K, C), lambda i: (i, 0, 0)))
        out_shapes.append(jax.ShapeDtypeStruct((N, _K, C), jnp.float32))
    if want_minmax:
        for _ in range(2):
            out_specs.append(pl.BlockSpec((BR, C), lambda i: (i, 0)))
            out_shapes.append(jax.ShapeDtypeStruct((N, C), jnp.float32))
    for _ in range(2):
        out_specs.append(pl.BlockSpec((1, 1, C), lambda i: (i, 0, 0)))
        out_shapes.append(jax.ShapeDtypeStruct((nb, 1, C), jnp.float32))

    return pl.pallas_call(
        body,
        grid=(nb,),
        in_specs=[
            pl.BlockSpec((BR, F), lambda i: (i, 0)),
            pl.BlockSpec((BR, _K, F), lambda i: (i, 0, 0)),
            pl.BlockSpec(W.shape, lambda i: (0, 0)),
            pl.BlockSpec((1, C), lambda i: (0, 0)),
        ],
        out_specs=out_specs,
        out_shape=out_shapes,
        interpret=_IT,
    )(x, xj3, W, b)


def _dense_relu(X, a, c, W, b, want_minmax):
    if want_minmax:
        N, K, Ci = X.shape
        BR = 128
    else:
        R, Ci = X.shape
        BR = 2048
        nb = R // BR
    Co = W.shape[1]

    if want_minmax:
        nb = N // BR

        def body(x_ref, a_ref, c_ref, w_ref, b_ref,
                 mx_ref, mn_ref, sum_ref, sq_ref):
            aa = a_ref[:]
            cc = c_ref[:]
            w = w_ref[:]
            bb = b_ref[:]
            s = jnp.zeros((BR, Co), jnp.float32)
            s2 = jnp.zeros((BR, Co), jnp.float32)
            mx = mn = None
            for kk in range(K):
                xn = x_ref[:, kk, :] * aa + cc
                h = jnp.maximum(
                    jnp.dot(xn, w, preferred_element_type=jnp.float32,
                            precision=lax.Precision.DEFAULT) + bb, 0.0)
                mx = h if kk == 0 else jnp.maximum(mx, h)
                mn = h if kk == 0 else jnp.minimum(mn, h)
                s = s + h
                s2 = s2 + h * h
            mx_ref[:] = mx
            mn_ref[:] = mn
            sum_ref[0] = jnp.sum(s, axis=0, keepdims=True)
            sq_ref[0] = jnp.sum(s2, axis=0, keepdims=True)

        return pl.pallas_call(
            body,
            grid=(nb,),
            in_specs=[
                pl.BlockSpec((BR, K, Ci), lambda i: (i, 0, 0)),
                pl.BlockSpec((1, Ci), lambda i: (0, 0)),
                pl.BlockSpec((1, Ci), lambda i: (0, 0)),
                pl.BlockSpec(W.shape, lambda i: (0, 0)),
                pl.BlockSpec((1, Co), lambda i: (0, 0)),
            ],
            out_specs=[
                pl.BlockSpec((BR, Co), lambda i: (i, 0)),
                pl.BlockSpec((BR, Co), lambda i: (i, 0)),
                pl.BlockSpec((1, 1, Co), lambda i: (i, 0, 0)),
                pl.BlockSpec((1, 1, Co), lambda i: (i, 0, 0)),
            ],
            out_shape=[
                jax.ShapeDtypeStruct((N, Co), jnp.float32),
                jax.ShapeDtypeStruct((N, Co), jnp.float32),
                jax.ShapeDtypeStruct((nb, 1, Co), jnp.float32),
                jax.ShapeDtypeStruct((nb, 1, Co), jnp.float32),
            ],
            interpret=_IT,
        )(X, a, c, W, b)

    def body(x_ref, a_ref, c_ref, w_ref, b_ref, h_ref, sum_ref, sq_ref):
        xn = x_ref[:] * a_ref[:] + c_ref[:]
        h = jnp.maximum(
            jnp.dot(xn, w_ref[:], preferred_element_type=jnp.float32,
                    precision=lax.Precision.DEFAULT) + b_ref[:], 0.0)
        h_ref[:] = h
        sum_ref[0] = jnp.sum(h, axis=0, keepdims=True)
        sq_ref[0] = jnp.sum(h * h, axis=0, keepdims=True)

    return pl.pallas_call(
        body,
        grid=(nb,),
        in_specs=[
            pl.BlockSpec((BR, Ci), lambda i: (i, 0)),
            pl.BlockSpec((1, Ci), lambda i: (0, 0)),
            pl.BlockSpec((1, Ci), lambda i: (0, 0)),
            pl.BlockSpec(W.shape, lambda i: (0, 0)),
            pl.BlockSpec((1, Co), lambda i: (0, 0)),
        ],
        out_specs=[
            pl.BlockSpec((BR, Co), lambda i: (i, 0)),
            pl.BlockSpec((1, 1, Co), lambda i: (i, 0, 0)),
            pl.BlockSpec((1, 1, Co), lambda i: (i, 0, 0)),
        ],
        out_shape=[
            jax.ShapeDtypeStruct((R, Co), jnp.float32),
            jax.ShapeDtypeStruct((nb, 1, Co), jnp.float32),
            jax.ShapeDtypeStruct((nb, 1, Co), jnp.float32),
        ],
        interpret=_IT,
    )(X, a, c, W, b)


def _affine_sel(mx, mn, a, c):
    N, C = mx.shape
    BR = 1024

    def body(mx_ref, mn_ref, a_ref, c_ref, o_ref):
        a = a_ref[:]
        o_ref[:] = jnp.where(a > 0, a * mx_ref[:], a * mn_ref[:]) + c_ref[:]

    return pl.pallas_call(
        body,
        grid=(N // BR,),
        in_specs=[
            pl.BlockSpec((BR, C), lambda i: (i, 0)),
            pl.BlockSpec((BR, C), lambda i: (i, 0)),
            pl.BlockSpec((1, C), lambda i: (0, 0)),
            pl.BlockSpec((1, C), lambda i: (0, 0)),
        ],
        out_specs=pl.BlockSpec((BR, C), lambda i: (i, 0)),
        out_shape=jax.ShapeDtypeStruct((N, C), jnp.float32),
        interpret=_IT,
    )(mx, mn, a, c)


def _lin1(x1, x2, batch_col, Wa, Wb, b):
    N = x1.shape[0]
    C = Wa.shape[1]
    BR = 256
    nb = N // BR

    def body(x1_ref, x2_ref, bat_ref, wa_ref, wb_ref, b_ref,
             sum_ref, sq_ref, mx_ref, mn_ref, mxs, mns):
        h = jnp.dot(x1_ref[:], wa_ref[:], preferred_element_type=jnp.float32,
                    precision=lax.Precision.DEFAULT)
        h = h + jnp.dot(x2_ref[:], wb_ref[:],
                        preferred_element_type=jnp.float32,
                        precision=lax.Precision.DEFAULT)
        h = jnp.maximum(h + b_ref[:], 0.0)
        sum_ref[0] = jnp.sum(h, axis=0, keepdims=True)
        sq_ref[0] = jnp.sum(h * h, axis=0, keepdims=True)
        bat = bat_ref[:]
        for g in range(_G):
            m = bat == g
            mxs[g:g + 1, :] = jnp.max(jnp.where(m, h, _NEG), axis=0,
                                      keepdims=True)
            mns[g:g + 1, :] = jnp.min(jnp.where(m, h, _POS), axis=0,
                                      keepdims=True)
        mx_ref[0] = mxs[:]
        mn_ref[0] = mns[:]

    return pl.pallas_call(
        body,
        grid=(nb,),
        in_specs=[
            pl.BlockSpec((BR, x1.shape[1]), lambda i: (i, 0)),
            pl.BlockSpec((BR, x2.shape[1]), lambda i: (i, 0)),
            pl.BlockSpec((BR, 1), lambda i: (i, 0)),
            pl.BlockSpec(Wa.shape, lambda i: (0, 0)),
            pl.BlockSpec(Wb.shape, lambda i: (0, 0)),
            pl.BlockSpec((1, C), lambda i: (0, 0)),
        ],
        out_specs=[
            pl.BlockSpec((1, 1, C), lambda i: (i, 0, 0)),
            pl.BlockSpec((1, 1, C), lambda i: (i, 0, 0)),
            pl.BlockSpec((1, _G, C), lambda i: (i, 0, 0)),
            pl.BlockSpec((1, _G, C), lambda i: (i, 0, 0)),
        ],
        out_shape=[
            jax.ShapeDtypeStruct((nb, 1, C), jnp.float32),
            jax.ShapeDtypeStruct((nb, 1, C), jnp.float32),
            jax.ShapeDtypeStruct((nb, _G, C), jnp.float32),
            jax.ShapeDtypeStruct((nb, _G, C), jnp.float32),
        ],
        scratch_shapes=[pltpu.VMEM((_G, C), jnp.float32),
                        pltpu.VMEM((_G, C), jnp.float32)],
        interpret=_IT,
    )(x1, x2, batch_col, Wa, Wb, b)


def _tail(mxP, mnP, sumP, sqP, g5, be5, p6, p7, Wf, bf):
    nb, G, C = mxP.shape
    n = float(_N)

    def body(mxP_ref, mnP_ref, sumP_ref, sqP_ref, g5_ref, be5_ref,
             W6_ref, b6_ref, g6_ref, be6_ref,
             W7_ref, b7_ref, g7_ref, be7_ref, Wf_ref, bf_ref, o_ref):
        s = sumP_ref[0]
        s2 = sqP_ref[0]
        MX = mxP_ref[0]
        MN = mnP_ref[0]
        for i in range(1, nb):
            s = s + sumP_ref[i]
            s2 = s2 + sqP_ref[i]
            MX = jnp.maximum(MX, mxP_ref[i])
            MN = jnp.minimum(MN, mnP_ref[i])
        m = s / n
        v = s2 / n - m * m
        aL = g5_ref[:] / jnp.sqrt(v + 1e-5)
        bL = be5_ref[:] - m * aL
        pooled = jnp.where(aL > 0, aL * MX, aL * MN) + bL

        def block(x, W_ref, b_ref, g_ref, be_ref):
            h = jnp.maximum(
                jnp.dot(x, W_ref[:], preferred_element_type=jnp.float32,
                        precision=lax.Precision.DEFAULT) + b_ref[:], 0.0)
            mu = jnp.mean(h, axis=0, keepdims=True)
            va = jnp.mean((h - mu) * (h - mu), axis=0, keepdims=True)
            return (h - mu) / jnp.sqrt(va + 1e-5) * g_ref[:] + be_ref[:]

        h1 = block(pooled, W6_ref, b6_ref, g6_ref, be6_ref)
        h2 = block(h1, W7_ref, b7_ref, g7_ref, be7_ref)
        logits = jnp.dot(h2, Wf_ref[:], preferred_element_type=jnp.float32,
                         precision=lax.Precision.DEFAULT) + bf_ref[:]
        z = logits - jnp.max(logits, axis=1, keepdims=True)
        o_ref[:] = z - jnp.log(jnp.sum(jnp.exp(z), axis=1, keepdims=True))

    ins = [mxP, mnP, sumP, sqP, g5, be5,
           p6["W"], p6["b"][None, :], p6["g"][None, :], p6["be"][None, :],
           p7["W"], p7["b"][None, :], p7["g"][None, :], p7["be"][None, :],
           Wf, bf]
    in_specs = [pl.BlockSpec(a.shape, lambda i, nd=a.ndim: (0,) * nd)
                for a in ins]

    return pl.pallas_call(
        body,
        grid=(1,),
        in_specs=in_specs,
        out_specs=pl.BlockSpec((G, 40), lambda i: (0, 0)),
        out_shape=jax.ShapeDtypeStruct((G, 40), jnp.float32),
        interpret=_IT,
    )(*ins)


def _bn_affine(sumP, sqP, n, g, be):
    s = jnp.sum(sumP, axis=(0, 1))
    s2 = jnp.sum(sqP, axis=(0, 1))
    m = s / n
    v = s2 / n - m * m
    a = g / jnp.sqrt(v + 1e-5)
    return a[None, :], (be - m * a)[None, :]


def kernel(pos, batch, params):
    N = pos.shape[0]
    batch_col = batch.astype(jnp.int32).reshape(N, 1)
    batch_rowT = batch.astype(jnp.int32).reshape(1, N)
    nE = jnp.float32(N * _K)

    pos16 = jnp.concatenate([pos, jnp.zeros((N, 13), jnp.float32)], axis=1)
    lohi = _knn_ranges(batch, N)
    idx1 = _knn(pos16[:, :8], batch_col, batch_rowT, lohi)
    c1 = params["conv1"]
    W1p = jnp.zeros((32, 64), jnp.float32)
    W1p = W1p.at[0:3].set(c1[0]["W"][:3]).at[16:19].set(c1[0]["W"][3:])
    posj = _sc_gather(pos16, idx1.reshape(-1))
    H1, s1, q1 = _edge_l1(pos16, posj.reshape(N, _K, -1), W1p,
                          c1[0]["b"][None, :], want_h=True, want_minmax=False)
    a1, b1 = _bn_affine(s1, q1, nE, c1[0]["g"], c1[0]["be"])
    H2, s2, q2 = _dense_relu(H1.reshape(N * _K, -1), a1, b1,
                             c1[1]["W"], c1[1]["b"][None, :],
                             want_minmax=False)
    a2, b2 = _bn_affine(s2, q2, nE, c1[1]["g"], c1[1]["be"])
    mx1, mn1, s3, q3 = _dense_relu(H2.reshape(N, _K, -1), a2, b2,
                                   c1[2]["W"], c1[2]["b"][None, :],
                                   want_minmax=True)
    a3, b3 = _bn_affine(s3, q3, nE, c1[2]["g"], c1[2]["be"])
    x1 = _affine_sel(mx1, mn1, a3, b3)

    idx2 = _knn(x1, batch_col, batch_rowT, lohi)
    c2p = params["conv2"][0]
    x1j = _sc_gather(x1, idx2.reshape(-1))
    mx2, mn2, s4, q4 = _edge_l1(x1, x1j.reshape(N, _K, -1), c2p["W"],
                                c2p["b"][None, :],
                                want_h=False, want_minmax=True)
    a4, b4 = _bn_affine(s4, q4, nE, c2p["g"], c2p["be"])
    x2 = _affine_sel(mx2, mn2, a4, b4)

    l1 = params["lin1"][0]
    Wa, Wb = l1["W"][:64], l1["W"][64:]
    sumP, sqP, mxP, mnP = _lin1(x1, x2, batch_col, Wa, Wb, l1["b"][None, :])
    return _tail(mxP, mnP, sumP, sqP, l1["g"][None, :], l1["be"][None, :],
                 params["mlp1"][0], params["mlp2"][0],
                 params["final"]["W"], params["final"]["b"][None, :])


# speedup vs baseline: 8.9467x; 1.5105x over previous
import functools

import jax
import jax.numpy as jnp
from jax import lax
from jax.experimental import pallas as pl
from jax.experimental.pallas import tpu as pltpu
from jax.experimental.pallas import tpu_sc as plsc

_N = 8192
_K = 20
_G = 16
_BIGMASK = 1e30
_NEG = -1e30
_POS = 1e30
_IT = False


def _knn(x, batch_col, batch_rowT, lohi):
    N, F = x.shape
    BR = 128
    W = 1024
    NS = N // W
    nb = N // BR

    def body(lohi_ref, xr_ref, bc_ref, xall_ref, brT_ref, idx_ref,
             dwin, cd, ci):
        i = pl.program_id(0)
        lo = lohi_ref[i, 0]
        hi = lohi_ref[i, 1]
        nwin = (hi - lo + (W - 1)) // W
        xr = xr_ref[:]
        sqr = jnp.sum(xr * xr, axis=1, keepdims=True)
        bc = bc_ref[:]

        @pl.when(nwin > 1)
        def _init():
            cd[:] = jnp.full((NS * BR, 32), float("inf"), jnp.float32)

        def win_body(w, carry):
            lo_w = pl.multiple_of(jnp.minimum(lo + w * W, N - W), 512)
            xw = xall_ref[pl.ds(lo_w, W), :]
            sqw = lax.dot_general(
                jnp.ones((1, F), jnp.float32), xw * xw,
                (((1,), (1,)), ((), ())),
                preferred_element_type=jnp.float32,
                precision=lax.Precision.HIGHEST)
            pw = lax.dot_general(
                xr, xw, (((1,), (1,)), ((), ())),
                preferred_element_type=jnp.float32,
                precision=lax.Precision.DEFAULT)
            dw = sqr + sqw - 2.0 * pw
            bw = brT_ref[:, pl.ds(lo_w, W)]
            dw = jnp.where(bc != bw, _BIGMASK, dw)
            io = lax.broadcasted_iota(jnp.int32, (BR, W), 1) + lo_w
            dwin[:] = dw
            for t in range(_K):
                dd = dwin[:]
                m = jnp.min(dd, axis=1, keepdims=True)
                amin = jnp.min(jnp.where(dd == m, io, 2**30),
                               axis=1, keepdims=True)
                cd[pl.ds(w * BR, BR), t:t + 1] = m
                ci[pl.ds(w * BR, BR), t:t + 1] = amin
                dwin[:] = jnp.where(io == amin, float("inf"), dd)
            return carry

        lax.fori_loop(0, nwin, win_body, 0)

        @pl.when(nwin == 1)
        def _fast():
            idx_ref[:, 0:_K] = ci[0:BR, 0:_K]

        @pl.when(nwin > 1)
        def _merge():
            sd = [cd[w * BR:(w + 1) * BR, :] for w in range(NS)]
            si = [ci[w * BR:(w + 1) * BR, :] for w in range(NS)]
            for t in range(_K):
                m = jnp.min(sd[0], axis=1, keepdims=True)
                for w in range(1, NS):
                    m = jnp.minimum(m, jnp.min(sd[w], axis=1, keepdims=True))
                am = jnp.min(jnp.where(sd[0] == m, si[0], 2**30),
                             axis=1, keepdims=True)
                for w in range(1, NS):
                    am = jnp.minimum(
                        am, jnp.min(jnp.where(sd[w] == m, si[w], 2**30),
                                    axis=1, keepdims=True))
                idx_ref[:, t:t + 1] = am
                sd = [jnp.where(si[w] == am, float("inf"), sd[w])
                      for w in range(NS)]

    return pl.pallas_call(
        body,
        grid=(nb,),
        in_specs=[
            pl.BlockSpec(memory_space=pltpu.SMEM),
            pl.BlockSpec((BR, F), lambda i: (i, 0)),
            pl.BlockSpec((BR, 1), lambda i: (i, 0)),
            pl.BlockSpec((N, F), lambda i: (0, 0)),
            pl.BlockSpec((1, N), lambda i: (0, 0)),
        ],
        out_specs=pl.BlockSpec((BR, _K), lambda i: (i, 0)),
        out_shape=jax.ShapeDtypeStruct((N, _K), jnp.int32),
        scratch_shapes=[pltpu.VMEM((BR, W), jnp.float32),
                        pltpu.VMEM((NS * BR, 32), jnp.float32),
                        pltpu.VMEM((NS * BR, 32), jnp.int32)],
        interpret=_IT,
    )(lohi, x, batch_col, x, batch_rowT)


def _knn_ranges(batch, N, BR=128):
    b = batch.astype(jnp.int32)
    ga = jnp.arange(_G, dtype=jnp.int32)
    starts = jnp.searchsorted(b, ga, side="left").astype(jnp.int32)
    ends = jnp.searchsorted(b, ga, side="right").astype(jnp.int32)
    counts = ends - starts
    g_lo = b[0::BR]
    g_hi = b[BR - 1::BR]
    lo = starts[g_lo]
    hi = ends[g_hi]
    span = (ga[None, :] >= g_lo[:, None]) & (ga[None, :] <= g_hi[:, None])
    mc = jnp.min(jnp.where(span, counts[None, :], 2**30), axis=1)
    small = mc < _K
    lo = jnp.where(small, 0, lo) & ~511
    hi = jnp.where(small, N, hi)
    return jnp.stack([lo, hi], axis=1).astype(jnp.int32)


def _sc_gather(table, idx):
    V, D = table.shape
    E = idx.shape[0]
    NC, NS = 2, 16
    NW = NC * NS
    per = E // NW
    CH = 128
    nch = per // CH
    mesh = plsc.VectorSubcoreMesh(core_axis_name="c", subcore_axis_name="s",
                                  num_cores=NC, num_subcores=NS)

    @functools.partial(
        pl.kernel, mesh=mesh,
        out_type=jax.ShapeDtypeStruct((E, D), jnp.float32),
        scratch_types=[
            pltpu.VMEM((CH,), jnp.int32),
            pltpu.VMEM((CH, D), jnp.float32),
            pltpu.SemaphoreType.DMA,
        ],
        compiler_params=pltpu.CompilerParams(use_tc_tiling_on_sc=False),
        interpret=_IT,
    )
    def k(table_hbm, idx_hbm, out_hbm, idx_v, rows_v, sem):
        wid = lax.axis_index("s") * NC + lax.axis_index("c")
        base = wid * per

        def chunk(c, carry):
            off = base + c * CH
            pltpu.sync_copy(idx_hbm.at[pl.ds(off, CH)], idx_v)
            pltpu.async_copy(table_hbm.at[idx_v], rows_v, sem).wait()
            pltpu.sync_copy(rows_v, out_hbm.at[pl.ds(off, CH)])
            return carry

        lax.fori_loop(0, nch, chunk, 0)

    return k(table, idx)


def _edge_l1(x, xj3, W, b, want_h, want_minmax):
    N, F = x.shape
    C = W.shape[1]
    BR = 128
    nb = N // BR

    def body(x_ref, xj_ref, w_ref, b_ref, *refs):
        r = list(refs)
        h_ref = r.pop(0) if want_h else None
        if want_minmax:
            mx_ref = r.pop(0)
            mn_ref = r.pop(0)
        sum_ref, sq_ref = r
        xi = x_ref[:]
        w = w_ref[:]
        bb = b_ref[:]
        s = jnp.zeros((BR, C), jnp.float32)
        s2 = jnp.zeros((BR, C), jnp.float32)
        mx = mn = None
        for kk in range(_K):
            e = jnp.concatenate([xi, xj_ref[:, kk, :] - xi], axis=1)
            h = jnp.maximum(
                jnp.dot(e, w, preferred_element_type=jnp.float32,
                        precision=lax.Precision.DEFAULT) + bb, 0.0)
            if want_h:
                h_ref[:, kk, :] = h
            if want_minmax:
                mx = h if kk == 0 else jnp.maximum(mx, h)
                mn = h if kk == 0 else jnp.minimum(mn, h)
            s = s + h
            s2 = s2 + h * h
        if want_minmax:
            mx_ref[:] = mx
            mn_ref[:] = mn
        sum_ref[0] = jnp.sum(s, axis=0, keepdims=True)
        sq_ref[0] = jnp.sum(s2, axis=0, keepdims=True)

    out_specs = []
    out_shapes = []
    if want_h:
        out_specs.append(pl.BlockSpec((BR, _K, C), lambda i: (i, 0, 0)))
        out_shapes.append(jax.ShapeDtypeStruct((N, _K, C), jnp.float32))
    if want_minmax:
        for _ in range(2):
            out_specs.append(pl.BlockSpec((BR, C), lambda i: (i, 0)))
            out_shapes.append(jax.ShapeDtypeStruct((N, C), jnp.float32))
    for _ in range(2):
        out_specs.append(pl.BlockSpec((1, 1, C), lambda i: (i, 0, 0)))
        out_shapes.append(jax.ShapeDtypeStruct((nb, 1, C), jnp.float32))

    return pl.pallas_call(
        body,
        grid=(nb,),
        in_specs=[
            pl.BlockSpec((BR, F), lambda i: (i, 0)),
            pl.BlockSpec((BR, _K, F), lambda i: (i, 0, 0)),
            pl.BlockSpec(W.shape, lambda i: (0, 0)),
            pl.BlockSpec((1, C), lambda i: (0, 0)),
        ],
        out_specs=out_specs,
        out_shape=out_shapes,
        interpret=_IT,
    )(x, xj3, W, b)


def _dense_relu(X, a, c, W, b, want_minmax):
    if want_minmax:
        N, K, Ci = X.shape
        BR = 128
    else:
        R, Ci = X.shape
        BR = 2048
        nb = R // BR
    Co = W.shape[1]

    if want_minmax:
        nb = N // BR

        def body(x_ref, a_ref, c_ref, w_ref, b_ref,
                 mx_ref, mn_ref, sum_ref, sq_ref):
            aa = a_ref[:]
            cc = c_ref[:]
            w = w_ref[:]
            bb = b_ref[:]
            s = jnp.zeros((BR, Co), jnp.float32)
            s2 = jnp.zeros((BR, Co), jnp.float32)
            mx = mn = None
            for kk in range(K):
                xn = x_ref[:, kk, :] * aa + cc
                h = jnp.maximum(
                    jnp.dot(xn, w, preferred_element_type=jnp.float32,
                            precision=lax.Precision.DEFAULT) + bb, 0.0)
                mx = h if kk == 0 else jnp.maximum(mx, h)
                mn = h if kk == 0 else jnp.minimum(mn, h)
                s = s + h
                s2 = s2 + h * h
            mx_ref[:] = mx
            mn_ref[:] = mn
            sum_ref[0] = jnp.sum(s, axis=0, keepdims=True)
            sq_ref[0] = jnp.sum(s2, axis=0, keepdims=True)

        return pl.pallas_call(
            body,
            grid=(nb,),
            in_specs=[
                pl.BlockSpec((BR, K, Ci), lambda i: (i, 0, 0)),
                pl.BlockSpec((1, Ci), lambda i: (0, 0)),
                pl.BlockSpec((1, Ci), lambda i: (0, 0)),
                pl.BlockSpec(W.shape, lambda i: (0, 0)),
                pl.BlockSpec((1, Co), lambda i: (0, 0)),
            ],
            out_specs=[
                pl.BlockSpec((BR, Co), lambda i: (i, 0)),
                pl.BlockSpec((BR, Co), lambda i: (i, 0)),
                pl.BlockSpec((1, 1, Co), lambda i: (i, 0, 0)),
                pl.BlockSpec((1, 1, Co), lambda i: (i, 0, 0)),
            ],
            out_shape=[
                jax.ShapeDtypeStruct((N, Co), jnp.float32),
                jax.ShapeDtypeStruct((N, Co), jnp.float32),
                jax.ShapeDtypeStruct((nb, 1, Co), jnp.float32),
                jax.ShapeDtypeStruct((nb, 1, Co), jnp.float32),
            ],
            interpret=_IT,
        )(X, a, c, W, b)

    def body(x_ref, a_ref, c_ref, w_ref, b_ref, h_ref, sum_ref, sq_ref):
        xn = x_ref[:] * a_ref[:] + c_ref[:]
        h = jnp.maximum(
            jnp.dot(xn, w_ref[:], preferred_element_type=jnp.float32,
                    precision=lax.Precision.DEFAULT) + b_ref[:], 0.0)
        h_ref[:] = h
        sum_ref[0] = jnp.sum(h, axis=0, keepdims=True)
        sq_ref[0] = jnp.sum(h * h, axis=0, keepdims=True)

    return pl.pallas_call(
        body,
        grid=(nb,),
        in_specs=[
            pl.BlockSpec((BR, Ci), lambda i: (i, 0)),
            pl.BlockSpec((1, Ci), lambda i: (0, 0)),
            pl.BlockSpec((1, Ci), lambda i: (0, 0)),
            pl.BlockSpec(W.shape, lambda i: (0, 0)),
            pl.BlockSpec((1, Co), lambda i: (0, 0)),
        ],
        out_specs=[
            pl.BlockSpec((BR, Co), lambda i: (i, 0)),
            pl.BlockSpec((1, 1, Co), lambda i: (i, 0, 0)),
            pl.BlockSpec((1, 1, Co), lambda i: (i, 0, 0)),
        ],
        out_shape=[
            jax.ShapeDtypeStruct((R, Co), jnp.float32),
            jax.ShapeDtypeStruct((nb, 1, Co), jnp.float32),
            jax.ShapeDtypeStruct((nb, 1, Co), jnp.float32),
        ],
        interpret=_IT,
    )(X, a, c, W, b)


def _affine_sel(mx, mn, a, c):
    N, C = mx.shape
    BR = 1024

    def body(mx_ref, mn_ref, a_ref, c_ref, o_ref):
        a = a_ref[:]
        o_ref[:] = jnp.where(a > 0, a * mx_ref[:], a * mn_ref[:]) + c_ref[:]

    return pl.pallas_call(
        body,
        grid=(N // BR,),
        in_specs=[
            pl.BlockSpec((BR, C), lambda i: (i, 0)),
            pl.BlockSpec((BR, C), lambda i: (i, 0)),
            pl.BlockSpec((1, C), lambda i: (0, 0)),
            pl.BlockSpec((1, C), lambda i: (0, 0)),
        ],
        out_specs=pl.BlockSpec((BR, C), lambda i: (i, 0)),
        out_shape=jax.ShapeDtypeStruct((N, C), jnp.float32),
        interpret=_IT,
    )(mx, mn, a, c)


def _lin1(x1, x2, batch_col, Wa, Wb, b):
    N = x1.shape[0]
    C = Wa.shape[1]
    BR = 256
    nb = N // BR

    def body(x1_ref, x2_ref, bat_ref, wa_ref, wb_ref, b_ref,
             sum_ref, sq_ref, mx_ref, mn_ref, mxs, mns):
        h = jnp.dot(x1_ref[:], wa_ref[:], preferred_element_type=jnp.float32,
                    precision=lax.Precision.DEFAULT)
        h = h + jnp.dot(x2_ref[:], wb_ref[:],
                        preferred_element_type=jnp.float32,
                        precision=lax.Precision.DEFAULT)
        h = jnp.maximum(h + b_ref[:], 0.0)
        sum_ref[0] = jnp.sum(h, axis=0, keepdims=True)
        sq_ref[0] = jnp.sum(h * h, axis=0, keepdims=True)
        bat = bat_ref[:]
        for g in range(_G):
            m = bat == g
            mxs[g:g + 1, :] = jnp.max(jnp.where(m, h, _NEG), axis=0,
                                      keepdims=True)
            mns[g:g + 1, :] = jnp.min(jnp.where(m, h, _POS), axis=0,
                                      keepdims=True)
        mx_ref[0] = mxs[:]
        mn_ref[0] = mns[:]

    return pl.pallas_call(
        body,
        grid=(nb,),
        in_specs=[
            pl.BlockSpec((BR, x1.shape[1]), lambda i: (i, 0)),
            pl.BlockSpec((BR, x2.shape[1]), lambda i: (i, 0)),
            pl.BlockSpec((BR, 1), lambda i: (i, 0)),
            pl.BlockSpec(Wa.shape, lambda i: (0, 0)),
            pl.BlockSpec(Wb.shape, lambda i: (0, 0)),
            pl.BlockSpec((1, C), lambda i: (0, 0)),
        ],
        out_specs=[
            pl.BlockSpec((1, 1, C), lambda i: (i, 0, 0)),
            pl.BlockSpec((1, 1, C), lambda i: (i, 0, 0)),
            pl.BlockSpec((1, _G, C), lambda i: (i, 0, 0)),
            pl.BlockSpec((1, _G, C), lambda i: (i, 0, 0)),
        ],
        out_shape=[
            jax.ShapeDtypeStruct((nb, 1, C), jnp.float32),
            jax.ShapeDtypeStruct((nb, 1, C), jnp.float32),
            jax.ShapeDtypeStruct((nb, _G, C), jnp.float32),
            jax.ShapeDtypeStruct((nb, _G, C), jnp.float32),
        ],
        scratch_shapes=[pltpu.VMEM((_G, C), jnp.float32),
                        pltpu.VMEM((_G, C), jnp.float32)],
        interpret=_IT,
    )(x1, x2, batch_col, Wa, Wb, b)


def _tail(mxP, mnP, sumP, sqP, g5, be5, p6, p7, Wf, bf):
    nb, G, C = mxP.shape
    n = float(_N)

    def body(mxP_ref, mnP_ref, sumP_ref, sqP_ref, g5_ref, be5_ref,
             W6_ref, b6_ref, g6_ref, be6_ref,
             W7_ref, b7_ref, g7_ref, be7_ref, Wf_ref, bf_ref, o_ref):
        s = sumP_ref[0]
        s2 = sqP_ref[0]
        MX = mxP_ref[0]
        MN = mnP_ref[0]
        for i in range(1, nb):
            s = s + sumP_ref[i]
            s2 = s2 + sqP_ref[i]
            MX = jnp.maximum(MX, mxP_ref[i])
            MN = jnp.minimum(MN, mnP_ref[i])
        m = s / n
        v = s2 / n - m * m
        aL = g5_ref[:] / jnp.sqrt(v + 1e-5)
        bL = be5_ref[:] - m * aL
        pooled = jnp.where(aL > 0, aL * MX, aL * MN) + bL

        def block(x, W_ref, b_ref, g_ref, be_ref):
            h = jnp.maximum(
                jnp.dot(x, W_ref[:], preferred_element_type=jnp.float32,
                        precision=lax.Precision.DEFAULT) + b_ref[:], 0.0)
            mu = jnp.mean(h, axis=0, keepdims=True)
            va = jnp.mean((h - mu) * (h - mu), axis=0, keepdims=True)
            return (h - mu) / jnp.sqrt(va + 1e-5) * g_ref[:] + be_ref[:]

        h1 = block(pooled, W6_ref, b6_ref, g6_ref, be6_ref)
        h2 = block(h1, W7_ref, b7_ref, g7_ref, be7_ref)
        logits = jnp.dot(h2, Wf_ref[:], preferred_element_type=jnp.float32,
                         precision=lax.Precision.DEFAULT) + bf_ref[:]
        z = logits - jnp.max(logits, axis=1, keepdims=True)
        o_ref[:] = z - jnp.log(jnp.sum(jnp.exp(z), axis=1, keepdims=True))

    ins = [mxP, mnP, sumP, sqP, g5, be5,
           p6["W"], p6["b"][None, :], p6["g"][None, :], p6["be"][None, :],
           p7["W"], p7["b"][None, :], p7["g"][None, :], p7["be"][None, :],
           Wf, bf]
    in_specs = [pl.BlockSpec(a.shape, lambda i, nd=a.ndim: (0,) * nd)
                for a in ins]

    return pl.pallas_call(
        body,
        grid=(1,),
        in_specs=in_specs,
        out_specs=pl.BlockSpec((G, 40), lambda i: (0, 0)),
        out_shape=jax.ShapeDtypeStruct((G, 40), jnp.float32),
        interpret=_IT,
    )(*ins)


def _bn_affine(sumP, sqP, n, g, be):
    s = jnp.sum(sumP, axis=(0, 1))
    s2 = jnp.sum(sqP, axis=(0, 1))
    m = s / n
    v = s2 / n - m * m
    a = g / jnp.sqrt(v + 1e-5)
    return a[None, :], (be - m * a)[None, :]


def kernel(pos, batch, params):
    N = pos.shape[0]
    batch_col = batch.astype(jnp.int32).reshape(N, 1)
    batch_rowT = batch.astype(jnp.int32).reshape(1, N)
    nE = jnp.float32(N * _K)

    pos16 = jnp.concatenate([pos, jnp.zeros((N, 13), jnp.float32)], axis=1)
    lohi = _knn_ranges(batch, N)
    idx1 = _knn(pos16[:, :8], batch_col, batch_rowT, lohi)
    c1 = params["conv1"]
    W1p = jnp.zeros((32, 64), jnp.float32)
    W1p = W1p.at[0:3].set(c1[0]["W"][:3]).at[16:19].set(c1[0]["W"][3:])
    posj = _sc_gather(pos16, idx1.reshape(-1))
    H1, s1, q1 = _edge_l1(pos16, posj.reshape(N, _K, -1), W1p,
                          c1[0]["b"][None, :], want_h=True, want_minmax=False)
    a1, b1 = _bn_affine(s1, q1, nE, c1[0]["g"], c1[0]["be"])
    H2, s2, q2 = _dense_relu(H1.reshape(N * _K, -1), a1, b1,
                             c1[1]["W"], c1[1]["b"][None, :],
                             want_minmax=False)
    a2, b2 = _bn_affine(s2, q2, nE, c1[1]["g"], c1[1]["be"])
    mx1, mn1, s3, q3 = _dense_relu(H2.reshape(N, _K, -1), a2, b2,
                                   c1[2]["W"], c1[2]["b"][None, :],
                                   want_minmax=True)
    a3, b3 = _bn_affine(s3, q3, nE, c1[2]["g"], c1[2]["be"])
    x1 = _affine_sel(mx1, mn1, a3, b3)

    idx2 = _knn(x1, batch_col, batch_rowT, lohi)
    c2p = params["conv2"][0]
    x1j = _sc_gather(x1, idx2.reshape(-1))
    mx2, mn2, s4, q4 = _edge_l1(x1, x1j.reshape(N, _K, -1), c2p["W"],
                                c2p["b"][None, :],
                                want_h=False, want_minmax=True)
    a4, b4 = _bn_affine(s4, q4, nE, c2p["g"], c2p["be"])
    x2 = _affine_sel(mx2, mn2, a4, b4)

    l1 = params["lin1"][0]
    Wa, Wb = l1["W"][:64], l1["W"][64:]
    sumP, sqP, mxP, mnP = _lin1(x1, x2, batch_col, Wa, Wb, l1["b"][None, :])
    return _tail(mxP, mnP, sumP, sqP, l1["g"][None, :], l1["be"][None, :],
                 params["mlp1"][0], params["mlp2"][0],
                 params["final"]["W"], params["final"]["b"][None, :])


# speedup vs baseline: 8.9570x; 1.0011x over previous
import functools

import jax
import jax.numpy as jnp
from jax import lax
from jax.experimental import pallas as pl
from jax.experimental.pallas import tpu as pltpu
from jax.experimental.pallas import tpu_sc as plsc

_N = 8192
_K = 20
_G = 16
_BIGMASK = 1e30
_NEG = -1e30
_POS = 1e30


def _knn(x, batch_col, batch_rowT, lohi):
    N, F = x.shape
    BR = 128
    W = 1024
    NS = N // W
    nb = N // BR

    def body(lohi_ref, xr_ref, bc_ref, xall_ref, brT_ref, idx_ref,
             dwin, cd, ci):
        i = pl.program_id(0)
        lo = lohi_ref[i, 0]
        hi = lohi_ref[i, 1]
        nwin = (hi - lo + (W - 1)) // W
        xr = xr_ref[:]
        sqr = jnp.sum(xr * xr, axis=1, keepdims=True)
        bc = bc_ref[:]

        @pl.when(nwin > 1)
        def _init():
            cd[:] = jnp.full((NS * BR, 32), float("inf"), jnp.float32)

        def win_body(w, carry):
            lo_w = pl.multiple_of(jnp.minimum(lo + w * W, N - W), 512)
            xw = xall_ref[pl.ds(lo_w, W), :]
            sqw = lax.dot_general(
                jnp.ones((1, F), jnp.float32), xw * xw,
                (((1,), (1,)), ((), ())),
                preferred_element_type=jnp.float32,
                precision=lax.Precision.HIGHEST)
            pw = lax.dot_general(
                xr, xw, (((1,), (1,)), ((), ())),
                preferred_element_type=jnp.float32,
                precision=lax.Precision.DEFAULT)
            dw = sqr + sqw - 2.0 * pw
            bw = brT_ref[:, pl.ds(lo_w, W)]
            dw = jnp.where(bc != bw, _BIGMASK, dw)
            io = lax.broadcasted_iota(jnp.int32, (BR, W), 1) + lo_w
            dwin[:] = dw
            for t in range(_K):
                dd = dwin[:]
                m = jnp.min(dd, axis=1, keepdims=True)
                amin = jnp.min(jnp.where(dd == m, io, 2**30),
                               axis=1, keepdims=True)
                cd[pl.ds(w * BR, BR), t:t + 1] = m
                ci[pl.ds(w * BR, BR), t:t + 1] = amin
                dwin[:] = jnp.where(io == amin, float("inf"), dd)
            return carry

        lax.fori_loop(0, nwin, win_body, 0)

        @pl.when(nwin == 1)
        def _fast():
            idx_ref[:, 0:_K] = ci[0:BR, 0:_K]

        @pl.when(nwin > 1)
        def _merge():
            sd = [cd[w * BR:(w + 1) * BR, :] for w in range(NS)]
            si = [ci[w * BR:(w + 1) * BR, :] for w in range(NS)]
            for t in range(_K):
                m = jnp.min(sd[0], axis=1, keepdims=True)
                for w in range(1, NS):
                    m = jnp.minimum(m, jnp.min(sd[w], axis=1, keepdims=True))
                am = jnp.min(jnp.where(sd[0] == m, si[0], 2**30),
                             axis=1, keepdims=True)
                for w in range(1, NS):
                    am = jnp.minimum(
                        am, jnp.min(jnp.where(sd[w] == m, si[w], 2**30),
                                    axis=1, keepdims=True))
                idx_ref[:, t:t + 1] = am
                sd = [jnp.where(si[w] == am, float("inf"), sd[w])
                      for w in range(NS)]

    return pl.pallas_call(
        body,
        grid=(nb,),
        in_specs=[
            pl.BlockSpec(memory_space=pltpu.SMEM),
            pl.BlockSpec((BR, F), lambda i: (i, 0)),
            pl.BlockSpec((BR, 1), lambda i: (i, 0)),
            pl.BlockSpec((N, F), lambda i: (0, 0)),
            pl.BlockSpec((1, N), lambda i: (0, 0)),
        ],
        out_specs=pl.BlockSpec((BR, _K), lambda i: (i, 0)),
        out_shape=jax.ShapeDtypeStruct((N, _K), jnp.int32),
        scratch_shapes=[pltpu.VMEM((BR, W), jnp.float32),
                        pltpu.VMEM((NS * BR, 32), jnp.float32),
                        pltpu.VMEM((NS * BR, 32), jnp.int32)],
    )(lohi, x, batch_col, x, batch_rowT)


def _knn_ranges(batch, N, BR=128):
    b = batch.astype(jnp.int32)
    ga = jnp.arange(_G, dtype=jnp.int32)
    starts = jnp.searchsorted(b, ga, side="left").astype(jnp.int32)
    ends = jnp.searchsorted(b, ga, side="right").astype(jnp.int32)
    counts = ends - starts
    g_lo = b[0::BR]
    g_hi = b[BR - 1::BR]
    lo = starts[g_lo]
    hi = ends[g_hi]
    span = (ga[None, :] >= g_lo[:, None]) & (ga[None, :] <= g_hi[:, None])
    mc = jnp.min(jnp.where(span, counts[None, :], 2**30), axis=1)
    small = mc < _K
    lo = jnp.where(small, 0, lo) & ~511
    hi = jnp.where(small, N, hi)
    return jnp.stack([lo, hi], axis=1).astype(jnp.int32)


def _sc_gather(table, idx):
    V, D = table.shape
    E = idx.shape[0]
    NC, NS = 2, 16
    NW = NC * NS
    per = E // NW
    CH = 128
    nch = per // CH
    mesh = plsc.VectorSubcoreMesh(core_axis_name="c", subcore_axis_name="s",
                                  num_cores=NC, num_subcores=NS)

    @functools.partial(
        pl.kernel, mesh=mesh,
        out_type=jax.ShapeDtypeStruct((E, D), jnp.float32),
        scratch_types=[
            pltpu.VMEM((CH,), jnp.int32),
            pltpu.VMEM((CH, D), jnp.float32),
            pltpu.SemaphoreType.DMA,
        ],
        compiler_params=pltpu.CompilerParams(use_tc_tiling_on_sc=False),
    )
    def k(table_hbm, idx_hbm, out_hbm, idx_v, rows_v, sem):
        wid = lax.axis_index("s") * NC + lax.axis_index("c")
        base = wid * per

        def chunk(c, carry):
            off = base + c * CH
            pltpu.sync_copy(idx_hbm.at[pl.ds(off, CH)], idx_v)
            pltpu.async_copy(table_hbm.at[idx_v], rows_v, sem).wait()
            pltpu.sync_copy(rows_v, out_hbm.at[pl.ds(off, CH)])
            return carry

        lax.fori_loop(0, nch, chunk, 0)

    return k(table, idx)


def _edge_l1(x, xj3, W, b, want_h, want_minmax):
    N, F = x.shape
    C = W.shape[1]
    BR = 128
    nb = N // BR

    def body(x_ref, xj_ref, w_ref, b_ref, *refs):
        r = list(refs)
        h_ref = r.pop(0) if want_h else None
        if want_minmax:
            mx_ref = r.pop(0)
            mn_ref = r.pop(0)
        sum_ref, sq_ref = r
        xi = x_ref[:]
        w = w_ref[:]
        bb = b_ref[:]
        s = jnp.zeros((BR, C), jnp.float32)
        s2 = jnp.zeros((BR, C), jnp.float32)
        mx = mn = None
        for kk in range(_K):
            e = jnp.concatenate([xi, xj_ref[:, kk, :] - xi], axis=1)
            h = jnp.maximum(
                jnp.dot(e, w, preferred_element_type=jnp.float32,
                        precision=lax.Precision.DEFAULT) + bb, 0.0)
            if want_h:
                h_ref[:, kk, :] = h
            if want_minmax:
                mx = h if kk == 0 else jnp.maximum(mx, h)
                mn = h if kk == 0 else jnp.minimum(mn, h)
            s = s + h
            s2 = s2 + h * h
        if want_minmax:
            mx_ref[:] = mx
            mn_ref[:] = mn
        sum_ref[0] = jnp.sum(s, axis=0, keepdims=True)
        sq_ref[0] = jnp.sum(s2, axis=0, keepdims=True)

    out_specs = []
    out_shapes = []
    if want_h:
        out_specs.append(pl.BlockSpec((BR, _K, C), lambda i: (i, 0, 0)))
        out_shapes.append(jax.ShapeDtypeStruct((N, _K, C), jnp.float32))
    if want_minmax:
        for _ in range(2):
            out_specs.append(pl.BlockSpec((BR, C), lambda i: (i, 0)))
            out_shapes.append(jax.ShapeDtypeStruct((N, C), jnp.float32))
    for _ in range(2):
        out_specs.append(pl.BlockSpec((1, 1, C), lambda i: (i, 0, 0)))
        out_shapes.append(jax.ShapeDtypeStruct((nb, 1, C), jnp.float32))

    return pl.pallas_call(
        body,
        grid=(nb,),
        in_specs=[
            pl.BlockSpec((BR, F), lambda i: (i, 0)),
            pl.BlockSpec((BR, _K, F), lambda i: (i, 0, 0)),
            pl.BlockSpec(W.shape, lambda i: (0, 0)),
            pl.BlockSpec((1, C), lambda i: (0, 0)),
        ],
        out_specs=out_specs,
        out_shape=out_shapes,
    )(x, xj3, W, b)


def _dense_relu(X, a, c, W, b, want_minmax):
    if want_minmax:
        N, K, Ci = X.shape
        BR = 128
    else:
        R, Ci = X.shape
        BR = 2048
        nb = R // BR
    Co = W.shape[1]

    if want_minmax:
        nb = N // BR

        def body(x_ref, a_ref, c_ref, w_ref, b_ref,
                 mx_ref, mn_ref, sum_ref, sq_ref):
            aa = a_ref[:]
            cc = c_ref[:]
            w = w_ref[:]
            bb = b_ref[:]
            s = jnp.zeros((BR, Co), jnp.float32)
            s2 = jnp.zeros((BR, Co), jnp.float32)
            mx = mn = None
            for kk in range(K):
                xn = x_ref[:, kk, :] * aa + cc
                h = jnp.maximum(
                    jnp.dot(xn, w, preferred_element_type=jnp.float32,
                            precision=lax.Precision.DEFAULT) + bb, 0.0)
                mx = h if kk == 0 else jnp.maximum(mx, h)
                mn = h if kk == 0 else jnp.minimum(mn, h)
                s = s + h
                s2 = s2 + h * h
            mx_ref[:] = mx
            mn_ref[:] = mn
            sum_ref[0] = jnp.sum(s, axis=0, keepdims=True)
            sq_ref[0] = jnp.sum(s2, axis=0, keepdims=True)

        return pl.pallas_call(
            body,
            grid=(nb,),
            in_specs=[
                pl.BlockSpec((BR, K, Ci), lambda i: (i, 0, 0)),
                pl.BlockSpec((1, Ci), lambda i: (0, 0)),
                pl.BlockSpec((1, Ci), lambda i: (0, 0)),
                pl.BlockSpec(W.shape, lambda i: (0, 0)),
                pl.BlockSpec((1, Co), lambda i: (0, 0)),
            ],
            out_specs=[
                pl.BlockSpec((BR, Co), lambda i: (i, 0)),
                pl.BlockSpec((BR, Co), lambda i: (i, 0)),
                pl.BlockSpec((1, 1, Co), lambda i: (i, 0, 0)),
                pl.BlockSpec((1, 1, Co), lambda i: (i, 0, 0)),
            ],
            out_shape=[
                jax.ShapeDtypeStruct((N, Co), jnp.float32),
                jax.ShapeDtypeStruct((N, Co), jnp.float32),
                jax.ShapeDtypeStruct((nb, 1, Co), jnp.float32),
                jax.ShapeDtypeStruct((nb, 1, Co), jnp.float32),
            ],
            )(X, a, c, W, b)

    def body(x_ref, a_ref, c_ref, w_ref, b_ref, h_ref, sum_ref, sq_ref):
        xn = x_ref[:] * a_ref[:] + c_ref[:]
        h = jnp.maximum(
            jnp.dot(xn, w_ref[:], preferred_element_type=jnp.float32,
                    precision=lax.Precision.DEFAULT) + b_ref[:], 0.0)
        h_ref[:] = h
        sum_ref[0] = jnp.sum(h, axis=0, keepdims=True)
        sq_ref[0] = jnp.sum(h * h, axis=0, keepdims=True)

    return pl.pallas_call(
        body,
        grid=(nb,),
        in_specs=[
            pl.BlockSpec((BR, Ci), lambda i: (i, 0)),
            pl.BlockSpec((1, Ci), lambda i: (0, 0)),
            pl.BlockSpec((1, Ci), lambda i: (0, 0)),
            pl.BlockSpec(W.shape, lambda i: (0, 0)),
            pl.BlockSpec((1, Co), lambda i: (0, 0)),
        ],
        out_specs=[
            pl.BlockSpec((BR, Co), lambda i: (i, 0)),
            pl.BlockSpec((1, 1, Co), lambda i: (i, 0, 0)),
            pl.BlockSpec((1, 1, Co), lambda i: (i, 0, 0)),
        ],
        out_shape=[
            jax.ShapeDtypeStruct((R, Co), jnp.float32),
            jax.ShapeDtypeStruct((nb, 1, Co), jnp.float32),
            jax.ShapeDtypeStruct((nb, 1, Co), jnp.float32),
        ],
    )(X, a, c, W, b)


def _affine_sel(mx, mn, a, c):
    N, C = mx.shape
    BR = 1024

    def body(mx_ref, mn_ref, a_ref, c_ref, o_ref):
        a = a_ref[:]
        o_ref[:] = jnp.where(a > 0, a * mx_ref[:], a * mn_ref[:]) + c_ref[:]

    return pl.pallas_call(
        body,
        grid=(N // BR,),
        in_specs=[
            pl.BlockSpec((BR, C), lambda i: (i, 0)),
            pl.BlockSpec((BR, C), lambda i: (i, 0)),
            pl.BlockSpec((1, C), lambda i: (0, 0)),
            pl.BlockSpec((1, C), lambda i: (0, 0)),
        ],
        out_specs=pl.BlockSpec((BR, C), lambda i: (i, 0)),
        out_shape=jax.ShapeDtypeStruct((N, C), jnp.float32),
    )(mx, mn, a, c)


def _lin1(x1, x2, batch_col, Wa, Wb, b):
    N = x1.shape[0]
    C = Wa.shape[1]
    BR = 256
    nb = N // BR

    def body(x1_ref, x2_ref, bat_ref, wa_ref, wb_ref, b_ref,
             sum_ref, sq_ref, mx_ref, mn_ref, mxs, mns):
        h = jnp.dot(x1_ref[:], wa_ref[:], preferred_element_type=jnp.float32,
                    precision=lax.Precision.DEFAULT)
        h = h + jnp.dot(x2_ref[:], wb_ref[:],
                        preferred_element_type=jnp.float32,
                        precision=lax.Precision.DEFAULT)
        h = jnp.maximum(h + b_ref[:], 0.0)
        sum_ref[0] = jnp.sum(h, axis=0, keepdims=True)
        sq_ref[0] = jnp.sum(h * h, axis=0, keepdims=True)
        bat = bat_ref[:]
        for g in range(_G):
            m = bat == g
            mxs[g:g + 1, :] = jnp.max(jnp.where(m, h, _NEG), axis=0,
                                      keepdims=True)
            mns[g:g + 1, :] = jnp.min(jnp.where(m, h, _POS), axis=0,
                                      keepdims=True)
        mx_ref[0] = mxs[:]
        mn_ref[0] = mns[:]

    return pl.pallas_call(
        body,
        grid=(nb,),
        in_specs=[
            pl.BlockSpec((BR, x1.shape[1]), lambda i: (i, 0)),
            pl.BlockSpec((BR, x2.shape[1]), lambda i: (i, 0)),
            pl.BlockSpec((BR, 1), lambda i: (i, 0)),
            pl.BlockSpec(Wa.shape, lambda i: (0, 0)),
            pl.BlockSpec(Wb.shape, lambda i: (0, 0)),
            pl.BlockSpec((1, C), lambda i: (0, 0)),
        ],
        out_specs=[
            pl.BlockSpec((1, 1, C), lambda i: (i, 0, 0)),
            pl.BlockSpec((1, 1, C), lambda i: (i, 0, 0)),
            pl.BlockSpec((1, _G, C), lambda i: (i, 0, 0)),
            pl.BlockSpec((1, _G, C), lambda i: (i, 0, 0)),
        ],
        out_shape=[
            jax.ShapeDtypeStruct((nb, 1, C), jnp.float32),
            jax.ShapeDtypeStruct((nb, 1, C), jnp.float32),
            jax.ShapeDtypeStruct((nb, _G, C), jnp.float32),
            jax.ShapeDtypeStruct((nb, _G, C), jnp.float32),
        ],
        scratch_shapes=[pltpu.VMEM((_G, C), jnp.float32),
                        pltpu.VMEM((_G, C), jnp.float32)],
    )(x1, x2, batch_col, Wa, Wb, b)


def _tail(mxP, mnP, sumP, sqP, g5, be5, p6, p7, Wf, bf):
    nb, G, C = mxP.shape
    n = float(_N)

    def body(mxP_ref, mnP_ref, sumP_ref, sqP_ref, g5_ref, be5_ref,
             W6_ref, b6_ref, g6_ref, be6_ref,
             W7_ref, b7_ref, g7_ref, be7_ref, Wf_ref, bf_ref, o_ref):
        s = sumP_ref[0]
        s2 = sqP_ref[0]
        MX = mxP_ref[0]
        MN = mnP_ref[0]
        for i in range(1, nb):
            s = s + sumP_ref[i]
            s2 = s2 + sqP_ref[i]
            MX = jnp.maximum(MX, mxP_ref[i])
            MN = jnp.minimum(MN, mnP_ref[i])
        m = s / n
        v = s2 / n - m * m
        aL = g5_ref[:] / jnp.sqrt(v + 1e-5)
        bL = be5_ref[:] - m * aL
        pooled = jnp.where(aL > 0, aL * MX, aL * MN) + bL

        def block(x, W_ref, b_ref, g_ref, be_ref):
            h = jnp.maximum(
                jnp.dot(x, W_ref[:], preferred_element_type=jnp.float32,
                        precision=lax.Precision.DEFAULT) + b_ref[:], 0.0)
            mu = jnp.mean(h, axis=0, keepdims=True)
            va = jnp.mean((h - mu) * (h - mu), axis=0, keepdims=True)
            return (h - mu) / jnp.sqrt(va + 1e-5) * g_ref[:] + be_ref[:]

        h1 = block(pooled, W6_ref, b6_ref, g6_ref, be6_ref)
        h2 = block(h1, W7_ref, b7_ref, g7_ref, be7_ref)
        logits = jnp.dot(h2, Wf_ref[:], preferred_element_type=jnp.float32,
                         precision=lax.Precision.DEFAULT) + bf_ref[:]
        z = logits - jnp.max(logits, axis=1, keepdims=True)
        o_ref[:] = z - jnp.log(jnp.sum(jnp.exp(z), axis=1, keepdims=True))

    ins = [mxP, mnP, sumP, sqP, g5, be5,
           p6["W"], p6["b"][None, :], p6["g"][None, :], p6["be"][None, :],
           p7["W"], p7["b"][None, :], p7["g"][None, :], p7["be"][None, :],
           Wf, bf]
    in_specs = [pl.BlockSpec(a.shape, lambda i, nd=a.ndim: (0,) * nd)
                for a in ins]

    return pl.pallas_call(
        body,
        grid=(1,),
        in_specs=in_specs,
        out_specs=pl.BlockSpec((G, 40), lambda i: (0, 0)),
        out_shape=jax.ShapeDtypeStruct((G, 40), jnp.float32),
    )(*ins)


def _bn_affine(sumP, sqP, n, g, be):
    s = jnp.sum(sumP, axis=(0, 1))
    s2 = jnp.sum(sqP, axis=(0, 1))
    m = s / n
    v = s2 / n - m * m
    a = g / jnp.sqrt(v + 1e-5)
    return a[None, :], (be - m * a)[None, :]


def kernel(pos, batch, params):
    N = pos.shape[0]
    batch_col = batch.astype(jnp.int32).reshape(N, 1)
    batch_rowT = batch.astype(jnp.int32).reshape(1, N)
    nE = jnp.float32(N * _K)

    pos16 = jnp.concatenate([pos, jnp.zeros((N, 13), jnp.float32)], axis=1)
    lohi = _knn_ranges(batch, N)
    idx1 = _knn(pos16[:, :8], batch_col, batch_rowT, lohi)
    c1 = params["conv1"]
    W1p = jnp.zeros((32, 64), jnp.float32)
    W1p = W1p.at[0:3].set(c1[0]["W"][:3]).at[16:19].set(c1[0]["W"][3:])
    posj = _sc_gather(pos16, idx1.reshape(-1))
    H1, s1, q1 = _edge_l1(pos16, posj.reshape(N, _K, -1), W1p,
                          c1[0]["b"][None, :], want_h=True, want_minmax=False)
    a1, b1 = _bn_affine(s1, q1, nE, c1[0]["g"], c1[0]["be"])
    H2, s2, q2 = _dense_relu(H1.reshape(N * _K, -1), a1, b1,
                             c1[1]["W"], c1[1]["b"][None, :],
                             want_minmax=False)
    a2, b2 = _bn_affine(s2, q2, nE, c1[1]["g"], c1[1]["be"])
    mx1, mn1, s3, q3 = _dense_relu(H2.reshape(N, _K, -1), a2, b2,
                                   c1[2]["W"], c1[2]["b"][None, :],
                                   want_minmax=True)
    a3, b3 = _bn_affine(s3, q3, nE, c1[2]["g"], c1[2]["be"])
    x1 = _affine_sel(mx1, mn1, a3, b3)

    idx2 = _knn(x1, batch_col, batch_rowT, lohi)
    c2p = params["conv2"][0]
    x1j = _sc_gather(x1, idx2.reshape(-1))
    mx2, mn2, s4, q4 = _edge_l1(x1, x1j.reshape(N, _K, -1), c2p["W"],
                                c2p["b"][None, :],
                                want_h=False, want_minmax=True)
    a4, b4 = _bn_affine(s4, q4, nE, c2p["g"], c2p["be"])
    x2 = _affine_sel(mx2, mn2, a4, b4)

    l1 = params["lin1"][0]
    Wa, Wb = l1["W"][:64], l1["W"][64:]
    sumP, sqP, mxP, mnP = _lin1(x1, x2, batch_col, Wa, Wb, l1["b"][None, :])
    return _tail(mxP, mnP, sumP, sqP, l1["g"][None, :], l1["be"][None, :],
                 params["mlp1"][0], params["mlp2"][0],
                 params["final"]["W"], params["final"]["b"][None, :])
